# baseline (speedup 1.0000x reference)
import functools

import jax
import jax.numpy as jnp
from jax import lax
from jax.experimental import pallas as pl
from jax.experimental.pallas import tpu as pltpu
from jax.experimental.pallas import tpu_sc as plsc

_N = 100000
_D = 64
_B = 16384
_NW = 32
_BPW = _B // _NW
_CB = 128
_NCH = _BPW // _CB

_C = 16384


def _tconcat_body(a_ref, b_ref, o_ref):
    o_ref[:, 0:_D] = a_ref[...].T
    o_ref[:, _D:2 * _D] = b_ref[...].T


def _transpose_concat(a_t, b_t):
    return pl.pallas_call(
        _tconcat_body,
        grid=(pl.cdiv(_N, _C),),
        in_specs=[pl.BlockSpec((_D, _C), lambda i: (0, i))] * 2,
        out_specs=pl.BlockSpec((_C, 2 * _D), lambda i: (i, 0)),
        out_shape=jax.ShapeDtypeStruct((_N, 2 * _D), jnp.float32),
    )(a_t, b_t)


_CC = 2048


def _cargo_body(g_ref, cdt_ref, o_ref):
    o_ref[0:128, :] = g_ref[...].T
    o_ref[128:192, :] = cdt_ref[...]


def _cargo_assemble(cargo_g, cargo_data_t):
    return pl.pallas_call(
        _cargo_body,
        grid=(_B // _CC,),
        in_specs=[pl.BlockSpec((_CC, 128), lambda i: (i, 0)),
                  pl.BlockSpec((_D, _CC), lambda i: (0, i))],
        out_specs=pl.BlockSpec((192, _CC), lambda i: (0, i)),
        out_shape=jax.ShapeDtypeStruct((192, _B), jnp.float32),
    )(cargo_g, cargo_data_t)


_sc_mesh = plsc.VectorSubcoreMesh(core_axis_name="c", subcore_axis_name="s")


def _wid():
    return lax.axis_index("s") * 2 + lax.axis_index("c")


def _sc_tgather_body(type_data_h, orders_types_h, cargo_types_h,
                     orders_data_h, orders_ref_h, cargo_g_out,
                     idx_a, idx_b, g_a, g_b, od, sem, sem2):
    base = _wid() * _BPW
    wrows = pl.ds(base, _BPW)
    od_in = pltpu.make_async_copy(orders_data_h.at[wrows], od, sem2)
    od_in.start()

    def chunk(c, carry):
        rows = pl.ds(base + c * _CB, _CB)
        w1 = [
            pltpu.make_async_copy(orders_types_h.at[rows], idx_a, sem),
            pltpu.make_async_copy(cargo_types_h.at[rows], idx_b, sem),
        ]
        for cp in w1:
            cp.start()
        for cp in w1:
            cp.wait()
        w2 = [
            pltpu.make_async_copy(type_data_h.at[idx_a], g_a, sem),
            pltpu.make_async_copy(type_data_h.at[idx_b], g_b, sem),
        ]
        for cp in w2:
            cp.start()
        for cp in w2:
            cp.wait()
        w3 = [
            pltpu.make_async_copy(g_a, orders_ref_h.at[rows, pl.ds(0, 128)], sem),
            pltpu.make_async_copy(g_b, cargo_g_out.at[rows], sem),
        ]
        for cp in w3:
            cp.start()
        for cp in w3:
            cp.wait()
        return carry

    lax.fori_loop(0, _NCH, chunk, 0)
    od_in.wait()
    od_out = pltpu.make_async_copy(
        od, orders_ref_h.at[wrows, pl.ds(256, 128)], sem2)
    od_out.start()
    od_out.wait()


_sc_tgather = functools.partial(
    pl.kernel,
    mesh=_sc_mesh,
    out_type=jax.ShapeDtypeStruct((_B, 128), jnp.float32),
    scratch_types=[
        pltpu.VMEM((_CB,), jnp.int32),
        pltpu.VMEM((_CB,), jnp.int32),
        pltpu.VMEM((_CB, 128), jnp.float32),
        pltpu.VMEM((_CB, 128), jnp.float32),
        pltpu.VMEM((_BPW, 128), jnp.float32),
        pltpu.SemaphoreType.DMA,
        pltpu.SemaphoreType.DMA,
    ],
)(_sc_tgather_body)


def _sc_sgather_body(sys_data_h, orders_systems_h, orders_ref_h,
                     idx, g, sem):
    base = _wid() * _BPW

    def chunk(c, carry):
        rows = pl.ds(base + c * _CB, _CB)
        cp = pltpu.make_async_copy(orders_systems_h.at[rows], idx, sem)
        cp.start()
        cp.wait()
        cp = pltpu.make_async_copy(sys_data_h.at[idx], g, sem)
        cp.start()
        cp.wait()
        cp = pltpu.make_async_copy(g, orders_ref_h.at[rows, pl.ds(128, 128)], sem)
        cp.start()
        cp.wait()
        return carry

    lax.fori_loop(0, _NCH, chunk, 0)


_sc_sgather = functools.partial(
    pl.kernel,
    mesh=_sc_mesh,
    out_type=(),
    scratch_types=[
        pltpu.VMEM((_CB,), jnp.int32),
        pltpu.VMEM((_CB, 128), jnp.float32),
        pltpu.SemaphoreType.DMA,
    ],
)(_sc_sgather_body)


def kernel(systems, system_notes, types, type_notes, orders_types,
           orders_systems, orders_data, cargo_types, cargo_data):
    orders_ref = jax.new_ref(lax.empty((_B, 384), jnp.float32))
    all_type_data = _transpose_concat(types.T, type_notes.T)
    cargo_gathered = _sc_tgather(all_type_data, orders_types, cargo_types,
                                 orders_data, orders_ref)
    all_system_data = _transpose_concat(systems.T, system_notes.T)
    _sc_sgather(all_system_data, orders_systems, orders_ref)
    cargo_t = _cargo_assemble(cargo_gathered, cargo_data.T)
    return (all_system_data, all_type_data, orders_ref[...], cargo_t.T)


# speedup vs baseline: 1.0352x; 1.0352x over previous
import functools

import jax
import jax.numpy as jnp
from jax import lax
from jax.experimental import pallas as pl
from jax.experimental.pallas import tpu as pltpu
from jax.experimental.pallas import tpu_sc as plsc

_N = 100000
_D = 64
_B = 16384
_NW = 32
_BPW = _B // _NW
_CB = 128
_NCH = _BPW // _CB

_C = 8192


def _tconcat_body(a_ref, b_ref, o_ref):
    o_ref[:, 0:_D] = a_ref[...].T
    o_ref[:, _D:2 * _D] = b_ref[...].T


def _transpose_concat(a_t, b_t):
    return pl.pallas_call(
        _tconcat_body,
        grid=(pl.cdiv(_N, _C),),
        in_specs=[pl.BlockSpec((_D, _C), lambda i: (0, i))] * 2,
        out_specs=pl.BlockSpec((_C, 2 * _D), lambda i: (i, 0)),
        out_shape=jax.ShapeDtypeStruct((_N, 2 * _D), jnp.float32),
    )(a_t, b_t)


_CC = 4096


def _cargo_body(g_ref, cdt_ref, o_ref):
    o_ref[0:128, :] = g_ref[...].T
    o_ref[128:192, :] = cdt_ref[...]


def _cargo_assemble(cargo_g, cargo_data_t):
    return pl.pallas_call(
        _cargo_body,
        grid=(_B // _CC,),
        in_specs=[pl.BlockSpec((_CC, 128), lambda i: (i, 0)),
                  pl.BlockSpec((_D, _CC), lambda i: (0, i))],
        out_specs=pl.BlockSpec((192, _CC), lambda i: (0, i)),
        out_shape=jax.ShapeDtypeStruct((192, _B), jnp.float32),
    )(cargo_g, cargo_data_t)


_sc_mesh = plsc.VectorSubcoreMesh(core_axis_name="c", subcore_axis_name="s")


def _wid():
    return lax.axis_index("s") * 2 + lax.axis_index("c")


def _sc_tgather_body(type_data_h, orders_types_h, cargo_types_h,
                     orders_data_h, orders_ref_h, cargo_g_out,
                     idx_a, idx_b, g_a, g_b, od, sem, sem2):
    base = _wid() * _BPW
    wrows = pl.ds(base, _BPW)
    od_in = pltpu.make_async_copy(orders_data_h.at[wrows], od, sem2)
    od_in.start()

    def chunk(c, carry):
        rows = pl.ds(base + c * _CB, _CB)
        w1 = [
            pltpu.make_async_copy(orders_types_h.at[rows], idx_a, sem),
            pltpu.make_async_copy(cargo_types_h.at[rows], idx_b, sem),
        ]
        for cp in w1:
            cp.start()
        for cp in w1:
            cp.wait()
        w2 = [
            pltpu.make_async_copy(type_data_h.at[idx_a], g_a, sem),
            pltpu.make_async_copy(type_data_h.at[idx_b], g_b, sem),
        ]
        for cp in w2:
            cp.start()
        for cp in w2:
            cp.wait()
        w3 = [
            pltpu.make_async_copy(g_a, orders_ref_h.at[rows, pl.ds(0, 128)], sem),
            pltpu.make_async_copy(g_b, cargo_g_out.at[rows], sem),
        ]
        for cp in w3:
            cp.start()
        for cp in w3:
            cp.wait()
        return carry

    lax.fori_loop(0, _NCH, chunk, 0)
    od_in.wait()
    od_out = pltpu.make_async_copy(
        od, orders_ref_h.at[wrows, pl.ds(256, 128)], sem2)
    od_out.start()
    od_out.wait()


_sc_tgather = functools.partial(
    pl.kernel,
    mesh=_sc_mesh,
    out_type=jax.ShapeDtypeStruct((_B, 128), jnp.float32),
    scratch_types=[
        pltpu.VMEM((_CB,), jnp.int32),
        pltpu.VMEM((_CB,), jnp.int32),
        pltpu.VMEM((_CB, 128), jnp.float32),
        pltpu.VMEM((_CB, 128), jnp.float32),
        pltpu.VMEM((_BPW, 128), jnp.float32),
        pltpu.SemaphoreType.DMA,
        pltpu.SemaphoreType.DMA,
    ],
)(_sc_tgather_body)


def _sc_sgather_body(sys_data_h, orders_systems_h, orders_ref_h,
                     idx0, idx1, g0, g1, si0, si1, sg, sw0, sw1):
    base = _wid() * _BPW
    idx = (idx0, idx1)
    g = (g0, g1)
    si = (si0, si1)
    sw = (sw0, sw1)

    def rows(c):
        return pl.ds(base + c * _CB, _CB)

    pltpu.make_async_copy(orders_systems_h.at[rows(0)], idx[0], si[0]).start()
    for c in range(_NCH):
        b = c & 1
        pltpu.make_async_copy(orders_systems_h.at[rows(c)], idx[b], si[b]).wait()
        if c >= 2:
            pltpu.make_async_copy(
                g[b], orders_ref_h.at[rows(c - 2), pl.ds(128, 128)], sw[b]).wait()
        gcp = pltpu.make_async_copy(sys_data_h.at[idx[b]], g[b], sg)
        gcp.start()
        if c + 1 < _NCH:
            pltpu.make_async_copy(
                orders_systems_h.at[rows(c + 1)], idx[1 - b], si[1 - b]).start()
        gcp.wait()
        pltpu.make_async_copy(
            g[b], orders_ref_h.at[rows(c), pl.ds(128, 128)], sw[b]).start()
    for c in (_NCH - 2, _NCH - 1):
        b = c & 1
        pltpu.make_async_copy(
            g[b], orders_ref_h.at[rows(c), pl.ds(128, 128)], sw[b]).wait()


_sc_sgather = functools.partial(
    pl.kernel,
    mesh=_sc_mesh,
    out_type=(),
    scratch_types=[
        pltpu.VMEM((_CB,), jnp.int32),
        pltpu.VMEM((_CB,), jnp.int32),
        pltpu.VMEM((_CB, 128), jnp.float32),
        pltpu.VMEM((_CB, 128), jnp.float32),
        pltpu.SemaphoreType.DMA,
        pltpu.SemaphoreType.DMA,
        pltpu.SemaphoreType.DMA,
        pltpu.SemaphoreType.DMA,
        pltpu.SemaphoreType.DMA,
    ],
)(_sc_sgather_body)


def kernel(systems, system_notes, types, type_notes, orders_types,
           orders_systems, orders_data, cargo_types, cargo_data):
    orders_ref = jax.new_ref(lax.empty((_B, 384), jnp.float32))
    all_type_data = _transpose_concat(types.T, type_notes.T)
    cargo_gathered = _sc_tgather(all_type_data, orders_types, cargo_types,
                                 orders_data, orders_ref)
    all_system_data = _transpose_concat(systems.T, system_notes.T)
    _sc_sgather(all_system_data, orders_systems, orders_ref)
    cargo_t = _cargo_assemble(cargo_gathered, cargo_data.T)
    return (all_system_data, all_type_data, orders_ref[...], cargo_t.T)


# speedup vs baseline: 1.0672x; 1.0309x over previous
import functools

import jax
import jax.numpy as jnp
from jax import lax
from jax.experimental import pallas as pl
from jax.experimental.pallas import tpu as pltpu
from jax.experimental.pallas import tpu_sc as plsc

_N = 100000
_D = 64
_B = 16384
_NW = 32
_BPW = _B // _NW
_CB = 128
_NCH = _BPW // _CB

_C = 10240


def _tconcat_body(a_ref, b_ref, o_ref):
    o_ref[:, 0:_D] = a_ref[...].T
    o_ref[:, _D:2 * _D] = b_ref[...].T


def _transpose_concat(a_t, b_t):
    return pl.pallas_call(
        _tconcat_body,
        grid=(pl.cdiv(_N, _C),),
        in_specs=[pl.BlockSpec((_D, _C), lambda i: (0, i))] * 2,
        out_specs=pl.BlockSpec((_C, 2 * _D), lambda i: (i, 0)),
        out_shape=jax.ShapeDtypeStruct((_N, 2 * _D), jnp.float32),
    )(a_t, b_t)


_CC = 4096


def _cargo_body(g_ref, cdt_ref, o_ref):
    o_ref[0:128, :] = g_ref[...].T
    o_ref[128:192, :] = cdt_ref[...]


def _cargo_assemble(cargo_g, cargo_data_t):
    return pl.pallas_call(
        _cargo_body,
        grid=(_B // _CC,),
        in_specs=[pl.BlockSpec((_CC, 128), lambda i: (i, 0)),
                  pl.BlockSpec((_D, _CC), lambda i: (0, i))],
        out_specs=pl.BlockSpec((192, _CC), lambda i: (0, i)),
        out_shape=jax.ShapeDtypeStruct((192, _B), jnp.float32),
    )(cargo_g, cargo_data_t)


_sc_mesh = plsc.VectorSubcoreMesh(core_axis_name="c", subcore_axis_name="s")


def _wid():
    return lax.axis_index("s") * 2 + lax.axis_index("c")


def _sc_tgather_body(type_data_h, orders_types_h, cargo_types_h,
                     orders_data_h, orders_ref_h, cargo_g_out,
                     idx_a, idx_b, g_a, g_b, od, sem, sem2):
    base = _wid() * _BPW
    wrows = pl.ds(base, _BPW)
    od_in = pltpu.make_async_copy(orders_data_h.at[wrows], od, sem2)
    od_in.start()

    def chunk(c, carry):
        rows = pl.ds(base + c * _CB, _CB)
        w1 = [
            pltpu.make_async_copy(orders_types_h.at[rows], idx_a, sem),
            pltpu.make_async_copy(cargo_types_h.at[rows], idx_b, sem),
        ]
        for cp in w1:
            cp.start()
        for cp in w1:
            cp.wait()
        w2 = [
            pltpu.make_async_copy(type_data_h.at[idx_a], g_a, sem),
            pltpu.make_async_copy(type_data_h.at[idx_b], g_b, sem),
        ]
        for cp in w2:
            cp.start()
        for cp in w2:
            cp.wait()
        w3 = [
            pltpu.make_async_copy(g_a, orders_ref_h.at[rows, pl.ds(0, 128)], sem),
            pltpu.make_async_copy(g_b, cargo_g_out.at[rows], sem),
        ]
        for cp in w3:
            cp.start()
        for cp in w3:
            cp.wait()
        return carry

    lax.fori_loop(0, _NCH, chunk, 0)
    od_in.wait()
    od_out = pltpu.make_async_copy(
        od, orders_ref_h.at[wrows, pl.ds(256, 128)], sem2)
    od_out.start()
    od_out.wait()


_sc_tgather = functools.partial(
    pl.kernel,
    mesh=_sc_mesh,
    out_type=jax.ShapeDtypeStruct((_B, 128), jnp.float32),
    scratch_types=[
        pltpu.VMEM((_CB,), jnp.int32),
        pltpu.VMEM((_CB,), jnp.int32),
        pltpu.VMEM((_CB, 128), jnp.float32),
        pltpu.VMEM((_CB, 128), jnp.float32),
        pltpu.VMEM((_BPW, 128), jnp.float32),
        pltpu.SemaphoreType.DMA,
        pltpu.SemaphoreType.DMA,
    ],
)(_sc_tgather_body)


def _sc_sgather_body(sys_data_h, orders_systems_h, orders_ref_h,
                     idx0, idx1, g0, g1, si0, si1, sg, sw0, sw1):
    base = _wid() * _BPW
    idx = (idx0, idx1)
    g = (g0, g1)
    si = (si0, si1)
    sw = (sw0, sw1)

    def rows(c):
        return pl.ds(base + c * _CB, _CB)

    pltpu.make_async_copy(orders_systems_h.at[rows(0)], idx[0], si[0]).start()
    for c in range(_NCH):
        b = c & 1
        pltpu.make_async_copy(orders_systems_h.at[rows(c)], idx[b], si[b]).wait()
        if c >= 2:
            pltpu.make_async_copy(
                g[b], orders_ref_h.at[rows(c - 2), pl.ds(128, 128)], sw[b]).wait()
        gcp = pltpu.make_async_copy(sys_data_h.at[idx[b]], g[b], sg)
        gcp.start()
        if c + 1 < _NCH:
            pltpu.make_async_copy(
                orders_systems_h.at[rows(c + 1)], idx[1 - b], si[1 - b]).start()
        gcp.wait()
        pltpu.make_async_copy(
            g[b], orders_ref_h.at[rows(c), pl.ds(128, 128)], sw[b]).start()
    for c in (_NCH - 2, _NCH - 1):
        b = c & 1
        pltpu.make_async_copy(
            g[b], orders_ref_h.at[rows(c), pl.ds(128, 128)], sw[b]).wait()


_sc_sgather = functools.partial(
    pl.kernel,
    mesh=_sc_mesh,
    out_type=(),
    scratch_types=[
        pltpu.VMEM((_CB,), jnp.int32),
        pltpu.VMEM((_CB,), jnp.int32),
        pltpu.VMEM((_CB, 128), jnp.float32),
        pltpu.VMEM((_CB, 128), jnp.float32),
        pltpu.SemaphoreType.DMA,
        pltpu.SemaphoreType.DMA,
        pltpu.SemaphoreType.DMA,
        pltpu.SemaphoreType.DMA,
        pltpu.SemaphoreType.DMA,
    ],
)(_sc_sgather_body)


def kernel(systems, system_notes, types, type_notes, orders_types,
           orders_systems, orders_data, cargo_types, cargo_data):
    orders_ref = jax.new_ref(lax.empty((_B, 384), jnp.float32))
    all_type_data = _transpose_concat(types.T, type_notes.T)
    cargo_gathered = _sc_tgather(all_type_data, orders_types, cargo_types,
                                 orders_data, orders_ref)
    all_system_data = _transpose_concat(systems.T, system_notes.T)
    _sc_sgather(all_system_data, orders_systems, orders_ref)
    cargo_t = _cargo_assemble(cargo_gathered, cargo_data.T)
    return (all_system_data, all_type_data, orders_ref[...], cargo_t.T)


# speedup vs baseline: 1.0695x; 1.0022x over previous
import functools

import jax
import jax.numpy as jnp
from jax import lax
from jax.experimental import pallas as pl
from jax.experimental.pallas import tpu as pltpu
from jax.experimental.pallas import tpu_sc as plsc

_N = 100000
_D = 64
_B = 16384
_NW = 32
_BPW = _B // _NW
_CB = 128
_NCH = _BPW // _CB

_C = 12800


def _tconcat_body(a_ref, b_ref, o_ref):
    o_ref[:, 0:_D] = a_ref[...].T
    o_ref[:, _D:2 * _D] = b_ref[...].T


def _transpose_concat(a_t, b_t):
    return pl.pallas_call(
        _tconcat_body,
        grid=(pl.cdiv(_N, _C),),
        in_specs=[pl.BlockSpec((_D, _C), lambda i: (0, i))] * 2,
        out_specs=pl.BlockSpec((_C, 2 * _D), lambda i: (i, 0)),
        out_shape=jax.ShapeDtypeStruct((_N, 2 * _D), jnp.float32),
    )(a_t, b_t)


_CC = 4096


def _cargo_body(g_ref, cdt_ref, o_ref):
    o_ref[0:128, :] = g_ref[...].T
    o_ref[128:192, :] = cdt_ref[...]


def _cargo_assemble(cargo_g, cargo_data_t):
    return pl.pallas_call(
        _cargo_body,
        grid=(_B // _CC,),
        in_specs=[pl.BlockSpec((_CC, 128), lambda i: (i, 0)),
                  pl.BlockSpec((_D, _CC), lambda i: (0, i))],
        out_specs=pl.BlockSpec((192, _CC), lambda i: (0, i)),
        out_shape=jax.ShapeDtypeStruct((192, _B), jnp.float32),
    )(cargo_g, cargo_data_t)


_sc_mesh = plsc.VectorSubcoreMesh(core_axis_name="c", subcore_axis_name="s")


def _wid():
    return lax.axis_index("s") * 2 + lax.axis_index("c")


def _sc_tgather_body(type_data_h, orders_types_h, cargo_types_h,
                     orders_data_h, orders_ref_h, cargo_g_out,
                     idx_a, idx_b, g_a, g_b, od, sem, sem2):
    base = _wid() * _BPW
    wrows = pl.ds(base, _BPW)
    od_in = pltpu.make_async_copy(orders_data_h.at[wrows], od, sem2)
    od_in.start()

    def chunk(c, carry):
        rows = pl.ds(base + c * _CB, _CB)
        w1 = [
            pltpu.make_async_copy(orders_types_h.at[rows], idx_a, sem),
            pltpu.make_async_copy(cargo_types_h.at[rows], idx_b, sem),
        ]
        for cp in w1:
            cp.start()
        for cp in w1:
            cp.wait()
        w2 = [
            pltpu.make_async_copy(type_data_h.at[idx_a], g_a, sem),
            pltpu.make_async_copy(type_data_h.at[idx_b], g_b, sem),
        ]
        for cp in w2:
            cp.start()
        for cp in w2:
            cp.wait()
        w3 = [
            pltpu.make_async_copy(g_a, orders_ref_h.at[rows, pl.ds(0, 128)], sem),
            pltpu.make_async_copy(g_b, cargo_g_out.at[rows], sem),
        ]
        for cp in w3:
            cp.start()
        for cp in w3:
            cp.wait()
        return carry

    lax.fori_loop(0, _NCH, chunk, 0)
    od_in.wait()
    od_out = pltpu.make_async_copy(
        od, orders_ref_h.at[wrows, pl.ds(256, 128)], sem2)
    od_out.start()
    od_out.wait()


_sc_tgather = functools.partial(
    pl.kernel,
    mesh=_sc_mesh,
    out_type=jax.ShapeDtypeStruct((_B, 128), jnp.float32),
    scratch_types=[
        pltpu.VMEM((_CB,), jnp.int32),
        pltpu.VMEM((_CB,), jnp.int32),
        pltpu.VMEM((_CB, 128), jnp.float32),
        pltpu.VMEM((_CB, 128), jnp.float32),
        pltpu.VMEM((_BPW, 128), jnp.float32),
        pltpu.SemaphoreType.DMA,
        pltpu.SemaphoreType.DMA,
    ],
)(_sc_tgather_body)


def _sc_sgather_body(sys_data_h, orders_systems_h, orders_ref_h,
                     idx0, idx1, g0, g1, si0, si1, sg, sw0, sw1):
    base = _wid() * _BPW
    idx = (idx0, idx1)
    g = (g0, g1)
    si = (si0, si1)
    sw = (sw0, sw1)

    def rows(c):
        return pl.ds(base + c * _CB, _CB)

    pltpu.make_async_copy(orders_systems_h.at[rows(0)], idx[0], si[0]).start()
    for c in range(_NCH):
        b = c & 1
        pltpu.make_async_copy(orders_systems_h.at[rows(c)], idx[b], si[b]).wait()
        if c >= 2:
            pltpu.make_async_copy(
                g[b], orders_ref_h.at[rows(c - 2), pl.ds(128, 128)], sw[b]).wait()
        gcp = pltpu.make_async_copy(sys_data_h.at[idx[b]], g[b], sg)
        gcp.start()
        if c + 1 < _NCH:
            pltpu.make_async_copy(
                orders_systems_h.at[rows(c + 1)], idx[1 - b], si[1 - b]).start()
        gcp.wait()
        pltpu.make_async_copy(
            g[b], orders_ref_h.at[rows(c), pl.ds(128, 128)], sw[b]).start()
    for c in (_NCH - 2, _NCH - 1):
        b = c & 1
        pltpu.make_async_copy(
            g[b], orders_ref_h.at[rows(c), pl.ds(128, 128)], sw[b]).wait()


_sc_sgather = functools.partial(
    pl.kernel,
    mesh=_sc_mesh,
    out_type=(),
    scratch_types=[
        pltpu.VMEM((_CB,), jnp.int32),
        pltpu.VMEM((_CB,), jnp.int32),
        pltpu.VMEM((_CB, 128), jnp.float32),
        pltpu.VMEM((_CB, 128), jnp.float32),
        pltpu.SemaphoreType.DMA,
        pltpu.SemaphoreType.DMA,
        pltpu.SemaphoreType.DMA,
        pltpu.SemaphoreType.DMA,
        pltpu.SemaphoreType.DMA,
    ],
)(_sc_sgather_body)


def kernel(systems, system_notes, types, type_notes, orders_types,
           orders_systems, orders_data, cargo_types, cargo_data):
    orders_ref = jax.new_ref(lax.empty((_B, 384), jnp.float32))
    all_type_data = _transpose_concat(types.T, type_notes.T)
    cargo_gathered = _sc_tgather(all_type_data, orders_types, cargo_types,
                                 orders_data, orders_ref)
    all_system_data = _transpose_concat(systems.T, system_notes.T)
    _sc_sgather(all_system_data, orders_systems, orders_ref)
    cargo_t = _cargo_assemble(cargo_gathered, cargo_data.T)
    return (all_system_data, all_type_data, orders_ref[...], cargo_t.T)
